# baseline (speedup 1.0000x reference)
import functools

import jax
import jax.numpy as jnp
from jax import lax
from jax.experimental import pallas as pl
from jax.experimental.pallas import tpu as pltpu
from jax.experimental.pallas import tpu_sc as plsc

NC = 2
NS = 16
NW = NC * NS


@functools.lru_cache(maxsize=None)
def _make_gather(batch: int, seq: int, hidden: int):
    assert batch == NW * 128 and hidden == 64 and seq % 2 == 0
    mesh = plsc.VectorSubcoreMesh(core_axis_name="c", subcore_axis_name="s")

    @functools.partial(
        pl.kernel,
        mesh=mesh,
        out_type=jax.ShapeDtypeStruct((seq, 8, NW, 8, 128), jnp.float32),
        scratch_types=[
            pltpu.VMEM((seq, 128), jnp.int32),
            pltpu.VMEM((2, 128), jnp.int32),
            pltpu.VMEM((2, 128), jnp.int32),
            pltpu.VMEM((128, 128), jnp.float32),
            pltpu.VMEM((128, 128), jnp.float32),
            pltpu.VMEM((8, 8, 128), jnp.float32),
            pltpu.VMEM((8, 8, 128), jnp.float32),
            pltpu.SemaphoreType.DMA,
            pltpu.SemaphoreType.DMA,
            pltpu.SemaphoreType.DMA,
        ],
        compiler_params=pltpu.CompilerParams(
            use_tc_tiling_on_sc=True, needs_layout_passes=False),
    )
    def gather(xt_hbm, tbl_hbm, out_hbm, idx_v, sbuf, pbuf,
               g0, g1, t0, t1, gs0, gs1, wsem):
        gbufs = (g0, g1)
        tbufs = (t0, t1)
        gsems = (gs0, gs1)
        wid = lax.axis_index("s") * NC + lax.axis_index("c")
        iota = jnp.arange(16, dtype=jnp.int32)

        pltpu.sync_copy(xt_hbm.at[:, pl.ds(wid * 128, 128)], idx_v)

        def prep(t, slot):
            for k in range(8):
                v = idx_v[t, pl.ds(16 * k, 16)]
                sbuf[slot, pl.ds(16 * k, 16)] = v >> 1
                pbuf[slot, pl.ds(16 * k, 16)] = (v & 1) * 64

        def fire(slot):
            pltpu.async_copy(tbl_hbm.at[sbuf.at[slot]], gbufs[slot],
                             gsems[slot])

        def drain_gather(slot):
            pltpu.make_async_copy(tbl_hbm.at[pl.ds(0, 128)], gbufs[slot],
                                  gsems[slot]).wait()

        def transpose(slot):
            g = gbufs[slot]
            tb = tbufs[slot]
            offs = [pbuf[slot, pl.ds(16 * k, 16)] for k in range(8)]

            def dbody(d2, carry):
                for dd in range(2):
                    dmod = (iota + (2 * d2 + dd)) & 15
                    for hh in range(4):
                        h_vec = dmod + 16 * hh
                        ht_vec = h_vec >> 3
                        hi_vec = h_vec & 7
                        for k in range(8):
                            rows = iota + 16 * k
                            cols = offs[k] + h_vec
                            vec = plsc.load_gather(g, [rows, cols])
                            plsc.store_scatter(tb, [ht_vec, hi_vec, rows],
                                               vec)
                return carry

            lax.fori_loop(0, 8, dbody, 0)

        def write(t, slot):
            for ht in range(8):
                pltpu.async_copy(tbufs[slot].at[ht],
                                 out_hbm.at[t, ht, wid], wsem)

        def drain_write(slot):
            for ht in range(8):
                pltpu.make_async_copy(tbl_hbm.at[pl.ds(0, 8)],
                                      tbufs[slot].at[ht], wsem).wait()

        prep(0, 0)
        fire(0)
        prep(1, 1)
        fire(1)

        def body(i, carry):
            for s in range(2):
                t = 2 * i + s
                drain_gather(s)
                transpose(s)
                write(t, s)
                prep(t + 2, s)
                fire(s)
                drain_write(s)
            return carry

        lax.fori_loop(0, seq // 2 - 1, body, 0)
        for s in range(2):
            t = seq - 2 + s
            drain_gather(s)
            transpose(s)
            write(t, s)
            drain_write(s)

    return gather


def kernel(x, table):
    batch, seq = x.shape
    hidden = table.shape[1]
    fn = _make_gather(batch, seq, hidden)
    t2 = jnp.reshape(table, (table.shape[0] // 2, 128))
    out5 = fn(jnp.transpose(x), t2)
    return out5.transpose(2, 4, 0, 1, 3).reshape(batch, seq, hidden)


# speedup vs baseline: 1.0833x; 1.0833x over previous
import functools

import jax
import jax.numpy as jnp
from jax import lax
from jax.experimental import pallas as pl
from jax.experimental.pallas import tpu as pltpu
from jax.experimental.pallas import tpu_sc as plsc

NC = 2
NS = 16
NW = NC * NS


@functools.lru_cache(maxsize=None)
def _make_gather(batch: int, seq: int, hidden: int):
    assert batch == NW * 128 and hidden == 64 and seq % 2 == 0
    mesh = plsc.VectorSubcoreMesh(core_axis_name="c", subcore_axis_name="s")

    @functools.partial(
        pl.kernel,
        mesh=mesh,
        out_type=jax.ShapeDtypeStruct((seq, 8, NW, 8, 128), jnp.float32),
        scratch_types=[
            pltpu.VMEM((seq, 128), jnp.int32),
            pltpu.VMEM((2, 128), jnp.int32),
            pltpu.VMEM((2, 128), jnp.int32),
            pltpu.VMEM((128, 128), jnp.float32),
            pltpu.VMEM((128, 128), jnp.float32),
            pltpu.VMEM((8, 8, 128), jnp.float32),
            pltpu.VMEM((8, 8, 128), jnp.float32),
            pltpu.SemaphoreType.DMA,
            pltpu.SemaphoreType.DMA,
            pltpu.SemaphoreType.DMA,
        ],
        compiler_params=pltpu.CompilerParams(
            use_tc_tiling_on_sc=True, needs_layout_passes=False),
    )
    def gather(xt_hbm, tbl_hbm, out_hbm, idx_v, sbuf, pbuf,
               g0, g1, t0, t1, gs0, gs1, wsem):
        gbufs = (g0, g1)
        tbufs = (t0, t1)
        gsems = (gs0, gs1)
        wid = lax.axis_index("s") * NC + lax.axis_index("c")
        iota = jnp.arange(16, dtype=jnp.int32)

        pltpu.sync_copy(xt_hbm.at[:, pl.ds(wid * 128, 128)], idx_v)

        def prep(t, slot):
            for k in range(8):
                v = idx_v[t, pl.ds(16 * k, 16)]
                sbuf[slot, pl.ds(16 * k, 16)] = v >> 1
                pbuf[slot, pl.ds(16 * k, 16)] = (v & 1) * 64

        def fire(slot):
            pltpu.async_copy(tbl_hbm.at[sbuf.at[slot]], gbufs[slot],
                             gsems[slot])

        def drain_gather(slot):
            pltpu.make_async_copy(tbl_hbm.at[pl.ds(0, 128)], gbufs[slot],
                                  gsems[slot]).wait()

        def transpose(slot):
            g = gbufs[slot]
            tb = tbufs[slot]
            offs = [pbuf[slot, pl.ds(16 * k, 16)] for k in range(8)]

            def dbody(d, carry):
                dmod = (iota + d) & 15
                for hh in range(4):
                    h_vec = dmod + 16 * hh
                    ht_vec = h_vec >> 3
                    hi_vec = h_vec & 7
                    for k in range(8):
                        rows = iota + 16 * k
                        cols = offs[k] + h_vec
                        vec = plsc.load_gather(g, [rows, cols])
                        plsc.store_scatter(tb, [ht_vec, hi_vec, rows], vec)
                return carry

            lax.fori_loop(0, 16, dbody, 0)

        def write(t, slot):
            for ht in range(8):
                pltpu.async_copy(tbufs[slot].at[ht],
                                 out_hbm.at[t, ht, wid], wsem)

        def drain_write(slot):
            for ht in range(8):
                pltpu.make_async_copy(tbl_hbm.at[pl.ds(0, 8)],
                                      tbufs[slot].at[ht], wsem).wait()

        prep(0, 0)
        fire(0)
        prep(1, 1)
        fire(1)
        for s in range(2):
            t = s
            drain_gather(s)
            transpose(s)
            write(t, s)
            prep(t + 2, s)
            fire(s)

        def body(i, carry):
            for s in range(2):
                t = 2 * i + s
                drain_gather(s)
                drain_write(s)
                transpose(s)
                write(t, s)
                prep(t + 2, s)
                fire(s)
            return carry

        lax.fori_loop(1, seq // 2 - 1, body, 0)
        for s in range(2):
            t = seq - 2 + s
            drain_gather(s)
            drain_write(s)
            transpose(s)
            write(t, s)
        for s in range(2):
            drain_write(s)

    return gather


def kernel(x, table):
    batch, seq = x.shape
    hidden = table.shape[1]
    fn = _make_gather(batch, seq, hidden)
    t2 = jnp.reshape(table, (table.shape[0] // 2, 128))
    out5 = fn(jnp.transpose(x), t2)
    return out5.transpose(2, 4, 0, 1, 3).reshape(batch, seq, hidden)


# speedup vs baseline: 1.1742x; 1.0839x over previous
import functools

import jax
import jax.numpy as jnp
from jax import lax
from jax.experimental import pallas as pl
from jax.experimental.pallas import tpu as pltpu
from jax.experimental.pallas import tpu_sc as plsc

NC = 2
NS = 16
NW = NC * NS


@functools.lru_cache(maxsize=None)
def _make_pack(vocab: int):
    n_full = vocab // 128
    per_tile = 246
    mesh = plsc.VectorSubcoreMesh(core_axis_name="c", subcore_axis_name="s")

    @functools.partial(
        pl.kernel,
        mesh=mesh,
        out_type=jax.ShapeDtypeStruct((vocab // 2, 128), jnp.float32),
        scratch_types=[
            pltpu.VMEM((64, 128), jnp.float32),
            pltpu.VMEM((64, 128), jnp.float32),
            pltpu.VMEM((64, 128), jnp.float32),
            pltpu.VMEM((64, 128), jnp.float32),
            pltpu.VMEM((32, 128), jnp.float32),
            pltpu.SemaphoreType.DMA,
            pltpu.SemaphoreType.DMA,
            pltpu.SemaphoreType.DMA,
        ],
        compiler_params=pltpu.CompilerParams(
            use_tc_tiling_on_sc=True, needs_layout_passes=False),
    )
    def pack(tt_hbm, tail_hbm, out_hbm, v0, v1, p0, p1, tl, s0, s1, wsem):
        vbs = (v0, v1)
        pbs = (p0, p1)
        sems = (s0, s1)
        wid = lax.axis_index("s") * NC + lax.axis_index("c")
        iota = jnp.arange(16, dtype=jnp.int32)

        @pl.when(wid == 0)
        def _():
            pltpu.sync_copy(tail_hbm, tl)
            pltpu.sync_copy(tl, out_hbm.at[pl.ds(n_full * 64, 32)])

        def jat(i):
            return jnp.minimum(wid * per_tile + i, n_full - 1)

        def stage(i, slot):
            j = jat(i)
            for ht in range(8):
                pltpu.async_copy(
                    tt_hbm.at[pl.ds(8 * ht, 8), pl.ds(128 * j, 128)],
                    vbs[slot].at[pl.ds(8 * ht, 8)], sems[slot])

        def drain_stage(slot):
            for ht in range(8):
                pltpu.make_async_copy(
                    tt_hbm.at[pl.ds(0, 8), pl.ds(0, 128)],
                    vbs[slot].at[pl.ds(8 * ht, 8)], sems[slot]).wait()

        def transpose(slot):
            vb = vbs[slot]
            pb = pbs[slot]

            def dbody(d, carry):
                dmod = (iota + d) & 15
                for c in range(8):
                    rows = dmod + 16 * (c % 4)
                    lvec = dmod + 16 * c
                    for a in range(4):
                        cols = 2 * iota + (32 * a + (1 if c >= 4 else 0))
                        vec = plsc.load_gather(vb, [rows, cols])
                        plsc.store_scatter(pb, [iota + 16 * a, lvec], vec)
                return carry

            lax.fori_loop(0, 16, dbody, 0)

        def write(i, slot):
            pltpu.async_copy(pbs[slot], out_hbm.at[pl.ds(64 * jat(i), 64)],
                             wsem)

        def drain_write(slot):
            pltpu.make_async_copy(tt_hbm.at[pl.ds(0, 64), pl.ds(0, 128)],
                                  pbs[slot], wsem).wait()

        stage(0, 0)
        stage(1, 1)
        for s in range(2):
            drain_stage(s)
            transpose(s)
            write(s, s)
            stage(s + 2, s)

        def body(i, carry):
            for s in range(2):
                u = 2 * i + s
                drain_stage(s)
                drain_write(s)
                transpose(s)
                write(u, s)
                stage(u + 2, s)
            return carry

        lax.fori_loop(1, per_tile // 2 - 1, body, 0)
        for s in range(2):
            u = per_tile - 2 + s
            drain_stage(s)
            drain_write(s)
            transpose(s)
            write(u, s)
        for s in range(2):
            drain_write(s)

    return pack


@functools.lru_cache(maxsize=None)
def _make_gather(batch: int, seq: int, hidden: int):
    assert batch == NW * 128 and hidden == 64 and seq % 2 == 0
    mesh = plsc.VectorSubcoreMesh(core_axis_name="c", subcore_axis_name="s")

    @functools.partial(
        pl.kernel,
        mesh=mesh,
        out_type=jax.ShapeDtypeStruct((seq, 8, NW, 8, 128), jnp.float32),
        scratch_types=[
            pltpu.VMEM((seq, 128), jnp.int32),
            pltpu.VMEM((2, 128), jnp.int32),
            pltpu.VMEM((2, 128), jnp.int32),
            pltpu.VMEM((128, 128), jnp.float32),
            pltpu.VMEM((128, 128), jnp.float32),
            pltpu.VMEM((8, 8, 128), jnp.float32),
            pltpu.VMEM((8, 8, 128), jnp.float32),
            pltpu.SemaphoreType.DMA,
            pltpu.SemaphoreType.DMA,
            pltpu.SemaphoreType.DMA,
        ],
        compiler_params=pltpu.CompilerParams(
            use_tc_tiling_on_sc=True, needs_layout_passes=False),
    )
    def gather(xt_hbm, tbl_hbm, out_hbm, idx_v, sbuf, pbuf,
               g0, g1, t0, t1, gs0, gs1, wsem):
        gbufs = (g0, g1)
        tbufs = (t0, t1)
        gsems = (gs0, gs1)
        wid = lax.axis_index("s") * NC + lax.axis_index("c")
        iota = jnp.arange(16, dtype=jnp.int32)

        pltpu.sync_copy(xt_hbm.at[:, pl.ds(wid * 128, 128)], idx_v)

        def prep(t, slot):
            for k in range(8):
                v = idx_v[t, pl.ds(16 * k, 16)]
                sbuf[slot, pl.ds(16 * k, 16)] = v >> 1
                pbuf[slot, pl.ds(16 * k, 16)] = (v & 1) * 64

        def fire(slot):
            pltpu.async_copy(tbl_hbm.at[sbuf.at[slot]], gbufs[slot],
                             gsems[slot])

        def drain_gather(slot):
            pltpu.make_async_copy(tbl_hbm.at[pl.ds(0, 128)], gbufs[slot],
                                  gsems[slot]).wait()

        def transpose(slot):
            g = gbufs[slot]
            tb = tbufs[slot]
            offs = [pbuf[slot, pl.ds(16 * k, 16)] for k in range(8)]

            def dbody(d, carry):
                dmod = (iota + d) & 15
                for hh in range(4):
                    h_vec = dmod + 16 * hh
                    ht_vec = h_vec >> 3
                    hi_vec = h_vec & 7
                    for k in range(8):
                        rows = iota + 16 * k
                        cols = offs[k] + h_vec
                        vec = plsc.load_gather(g, [rows, cols])
                        plsc.store_scatter(tb, [ht_vec, hi_vec, rows], vec)
                return carry

            lax.fori_loop(0, 16, dbody, 0)

        def write(t, slot):
            for ht in range(8):
                pltpu.async_copy(tbufs[slot].at[ht],
                                 out_hbm.at[t, ht, wid], wsem)

        def drain_write(slot):
            for ht in range(8):
                pltpu.make_async_copy(tbl_hbm.at[pl.ds(0, 8)],
                                      tbufs[slot].at[ht], wsem).wait()

        prep(0, 0)
        fire(0)
        prep(1, 1)
        fire(1)
        for s in range(2):
            t = s
            drain_gather(s)
            transpose(s)
            write(t, s)
            prep(t + 2, s)
            fire(s)

        def body(i, carry):
            for s in range(2):
                t = 2 * i + s
                drain_gather(s)
                drain_write(s)
                transpose(s)
                write(t, s)
                prep(t + 2, s)
                fire(s)
            return carry

        lax.fori_loop(1, seq // 2 - 1, body, 0)
        for s in range(2):
            t = seq - 2 + s
            drain_gather(s)
            drain_write(s)
            transpose(s)
            write(t, s)
        for s in range(2):
            drain_write(s)

    return gather


def kernel(x, table):
    batch, seq = x.shape
    vocab, hidden = table.shape
    pk = _make_pack(vocab)
    fn = _make_gather(batch, seq, hidden)
    n_tail = vocab - (vocab // 128) * 128
    tail = jnp.reshape(
        lax.slice(table, (vocab - n_tail, 0), (vocab, hidden)),
        (n_tail // 2, 128))
    t2 = pk(jnp.transpose(table), tail)
    out5 = fn(jnp.transpose(x), t2)
    return out5.transpose(2, 4, 0, 1, 3).reshape(batch, seq, hidden)
